# Initial kernel scaffold; baseline (speedup 1.0000x reference)
import jax
import jax.numpy as jnp
from jax.experimental import pallas as pl


def kernel(text, emb_table, W1, b1, W2, b2):
    raise NotImplementedError("write your pallas kernel here")


# speedup vs baseline: 17.1832x; 17.1832x over previous
import functools

import jax
import jax.numpy as jnp
from jax import lax
from jax.experimental import pallas as pl
from jax.experimental.pallas import tpu as pltpu
from jax.experimental.pallas import tpu_sc as plsc

B = 16384
L = 200
E = 32
H = 64
O = 2

NC = 2
NS = 16
NW = NC * NS
RPW = B // NW
GCHUNK = 128
NJ = RPW // GCHUNK
LAG = 8


def _sc_pool_sum(idx_all, emb_table):
    mesh = plsc.VectorSubcoreMesh(core_axis_name="c", subcore_axis_name="s")

    @functools.partial(
        pl.kernel,
        mesh=mesh,
        out_type=jax.ShapeDtypeStruct((B, E), jnp.float32),
        scratch_types=[
            pltpu.VMEM((L, RPW), jnp.int32),
            pltpu.VMEM((RPW, E), jnp.float32),
            pltpu.SemaphoreType.DMA,
        ],
        compiler_params=pltpu.CompilerParams(use_tc_tiling_on_sc=False),
    )
    def k(idx_hbm, table_hbm, out_hbm, idx_v, acc_v, gsem):
        cid = lax.axis_index("c")
        sid = lax.axis_index("s")
        wid = sid * NC + cid
        base = wid * RPW

        pltpu.sync_copy(idx_hbm.at[wid], idx_v)

        zero = jnp.zeros((16,), jnp.float32)

        def zbody(i, _):
            acc_v[i, pl.ds(0, 16)] = zero
            acc_v[i, pl.ds(16, 16)] = zero
            return 0

        lax.fori_loop(0, RPW, zbody, 0)

        def issue(t, j):
            return pltpu.async_copy(
                table_hbm.at[idx_v.at[t, pl.ds(j * GCHUNK, GCHUNK)]],
                acc_v.at[pl.ds(j * GCHUNK, GCHUNK)],
                gsem,
                add=True,
            )

        def drain(t, j):
            pltpu.make_async_copy(
                table_hbm.at[idx_v.at[t, pl.ds(j * GCHUNK, GCHUNK)]],
                acc_v.at[pl.ds(j * GCHUNK, GCHUNK)],
                gsem,
            ).wait()

        def gbody(t, _):
            for j in range(NJ):
                issue(t, j)

            @pl.when(t >= LAG)
            def _():
                for j in range(NJ):
                    drain(t - LAG, j)

            return 0

        lax.fori_loop(0, L, gbody, 0)
        for tt in range(L - LAG, L):
            for j in range(NJ):
                drain(tt, j)

        pltpu.sync_copy(acc_v, out_hbm.at[pl.ds(base, RPW)])

    return k(idx_all, emb_table)


def _mlp_kernel(x_ref, w1_ref, b1_ref, w2_ref, b2_ref, o_ref):
    x = x_ref[...] * (1.0 / L)
    h = jnp.dot(x, w1_ref[...], preferred_element_type=jnp.float32)
    h = jnp.maximum(h + b1_ref[...], 0.0)
    o = jnp.dot(h, w2_ref[...], preferred_element_type=jnp.float32)
    o = o + b2_ref[...]
    o_ref[...] = 1.0 / (1.0 + jnp.exp(-o))


def kernel(text, emb_table, W1, b1, W2, b2):
    idx_all = (
        text.astype(jnp.int32).T.reshape(L, NW, RPW).transpose(1, 0, 2)
    )

    pooled_sum = _sc_pool_sum(idx_all, emb_table)

    bt = 2048
    out = pl.pallas_call(
        _mlp_kernel,
        out_shape=jax.ShapeDtypeStruct((B, O), jnp.float32),
        grid=(B // bt,),
        in_specs=[
            pl.BlockSpec((bt, E), lambda i: (i, 0)),
            pl.BlockSpec((E, H), lambda i: (0, 0)),
            pl.BlockSpec((1, H), lambda i: (0, 0)),
            pl.BlockSpec((H, O), lambda i: (0, 0)),
            pl.BlockSpec((1, O), lambda i: (0, 0)),
        ],
        out_specs=pl.BlockSpec((bt, O), lambda i: (i, 0)),
    )(pooled_sum, W1.T, b1[None, :], W2.T, b2[None, :])
    return out
